# baseline (speedup 1.0000x reference)
import functools

import jax
import jax.numpy as jnp
from jax import lax
from jax.experimental import pallas as pl
from jax.experimental.pallas import tpu as pltpu
from jax.experimental.pallas import tpu_sc as plsc

N = 10000
E = 320000
F_IN = 128
HID = 64
G = 64
EPS = 1e-5

NC = 2
NS = 16
NW = NC * NS
EPW = E // NW
CH = 80
NCH = EPW // CH
NB = 5
RPT = N // NS
DW = 16

_mesh = functools.partial(
    plsc.VectorSubcoreMesh, core_axis_name="c", subcore_axis_name="s")

_f32 = jnp.float32
_i32 = jnp.int32


def _wid():
    return lax.axis_index("c") * NS + lax.axis_index("s")


def _deg_body(col_hbm, ew_hbm, z_hbm, out_hbm, col_s, ew_s, upd, deg_sh):
    cid = lax.axis_index("c")
    sid = lax.axis_index("s")
    wid = cid * NS + sid

    rsl = pl.ds(sid * RPT, RPT)
    pltpu.sync_copy(z_hbm.at[rsl], deg_sh.at[rsl])
    pltpu.sync_copy(col_hbm.at[wid], col_s)
    pltpu.sync_copy(ew_hbm.at[pl.ds(wid * EPW, EPW)], ew_s)
    plsc.subcore_barrier()

    def chunk(j, carry):
        def bld(g2, c2):
            n16 = ew_s[pl.ds(j * CH + g2 * 16, 16)]
            for lane in range(16):
                upd[g2 * 16 + lane, :] = jnp.full((DW,), n16[lane], _f32)
            return c2

        lax.fori_loop(0, CH // 16, bld, 0)
        pltpu.sync_copy(upd, deg_sh.at[col_s.at[j]], add=True)
        return carry

    lax.fori_loop(0, NCH, chunk, 0)
    plsc.subcore_barrier()

    @pl.when(sid == 0)
    def _():
        pltpu.sync_copy(deg_sh, out_hbm.at[cid])


_deg_call = pl.kernel(
    _deg_body,
    out_type=jax.ShapeDtypeStruct((NC, N, DW), _f32),
    mesh=_mesh(),
    compiler_params=pltpu.CompilerParams(use_tc_tiling_on_sc=False, needs_layout_passes=False),
    scratch_types=[
        pltpu.VMEM((NCH, CH), _i32),
        pltpu.VMEM((EPW,), _f32),
        pltpu.VMEM((CH, DW), _f32),
        pltpu.VMEM_SHARED((N, DW), _f32),
    ],
)


def _norm_body(row_hbm, col_hbm, ew_hbm, dis_hbm, out_hbm,
               row_s, col_s, ew_s, dis_s, nrm_s):
    wid = _wid()
    base = wid * EPW
    pltpu.sync_copy(row_hbm.at[pl.ds(base, EPW)], row_s)
    pltpu.sync_copy(col_hbm.at[pl.ds(base, EPW)], col_s)
    pltpu.sync_copy(ew_hbm.at[pl.ds(base, EPW)], ew_s)
    pltpu.sync_copy(dis_hbm, dis_s)

    def body(i, carry):
        sl = pl.ds(i * 16, 16)
        r = row_s[sl]
        c = col_s[sl]
        e = ew_s[sl]
        dr = plsc.load_gather(dis_s, [r])
        dc = plsc.load_gather(dis_s, [c])
        nrm_s[sl] = dr * e * dc
        return carry

    lax.fori_loop(0, EPW // 16, body, 0)
    pltpu.sync_copy(nrm_s, out_hbm.at[pl.ds(base, EPW)])


_norm_call = pl.kernel(
    _norm_body,
    out_type=jax.ShapeDtypeStruct((E,), _f32),
    mesh=_mesh(),
    compiler_params=pltpu.CompilerParams(use_tc_tiling_on_sc=False, needs_layout_passes=False),
    scratch_types=[
        pltpu.VMEM((EPW,), _i32),
        pltpu.VMEM((EPW,), _i32),
        pltpu.VMEM((EPW,), _f32),
        pltpu.VMEM((N,), _f32),
        pltpu.VMEM((EPW,), _f32),
    ],
)


def _agg_body(row_hbm, col_hbm, nrm_hbm, xw_hbm, z_hbm, out_hbm,
              row_s, col_s, nrm_s, rows_v, agg_sh, gsem, ssem):
    cid = lax.axis_index("c")
    sid = lax.axis_index("s")
    wid = cid * NS + sid

    rsl = pl.ds(sid * RPT, RPT)
    pltpu.sync_copy(z_hbm.at[rsl], agg_sh.at[rsl])
    pltpu.sync_copy(row_hbm.at[wid], row_s)
    pltpu.sync_copy(col_hbm.at[wid], col_s)
    pltpu.sync_copy(nrm_hbm.at[pl.ds(wid * EPW, EPW)], nrm_s)
    plsc.subcore_barrier()

    def group(gi, carry):
        gd = [pltpu.async_copy(xw_hbm.at[row_s.at[gi * NB + b]],
                               rows_v.at[pl.ds(b * CH, CH)], gsem)
              for b in range(NB)]
        sd = []
        for b in range(NB):
            j = gi * NB + b
            gd[b].wait()

            def scale(g2, c2, b=b, j=j):
                n16 = nrm_s[pl.ds(j * CH + g2 * 16, 16)]
                er = lax.iota(_i32, 16) + (b * CH + g2 * 16)
                ck = jnp.zeros((16,), _i32)
                for _ in range(HID):
                    v = plsc.load_gather(rows_v, [er, ck])
                    plsc.store_scatter(rows_v, [er, ck], v * n16)
                    ck = ck + 1
                return c2

            lax.fori_loop(0, CH // 16, scale, 0)
            sd.append(pltpu.async_copy(rows_v.at[pl.ds(b * CH, CH)],
                                       agg_sh.at[col_s.at[j]], ssem, add=True))
        for d in sd:
            d.wait()
        return carry

    lax.fori_loop(0, NCH // NB, group, 0)
    plsc.subcore_barrier()
    pltpu.sync_copy(agg_sh.at[rsl], out_hbm.at[cid].at[rsl])


_agg_call = pl.kernel(
    _agg_body,
    out_type=jax.ShapeDtypeStruct((NC, N, HID), _f32),
    mesh=_mesh(),
    compiler_params=pltpu.CompilerParams(use_tc_tiling_on_sc=False, needs_layout_passes=False),
    scratch_types=[
        pltpu.VMEM((NCH, CH), _i32),
        pltpu.VMEM((NCH, CH), _i32),
        pltpu.VMEM((EPW,), _f32),
        pltpu.VMEM((NB * CH, HID), _f32),
        pltpu.VMEM_SHARED((N, HID), _f32),
        pltpu.SemaphoreType.DMA,
        pltpu.SemaphoreType.DMA,
    ],
)


def _dot(a, b):
    return lax.dot_general(a, b, (((1,), (0,)), ((), ())),
                           precision=lax.Precision.HIGHEST,
                           preferred_element_type=_f32)


def _pre_body(degp, x, w1, dis_o, self_o, xw_o):
    deg = degp[0][:, 0:1] + degp[1][:, 0:1] + 1.0
    dis = jnp.where(deg > 0, lax.rsqrt(jnp.maximum(deg, 1e-12)), 0.0)
    dis_o[...] = dis
    self_o[...] = dis * dis
    xw_o[...] = _dot(x[...], w1[...])


_pre_call = pl.pallas_call(
    _pre_body,
    out_shape=[
        jax.ShapeDtypeStruct((N, 1), _f32),
        jax.ShapeDtypeStruct((N, 1), _f32),
        jax.ShapeDtypeStruct((N, HID), _f32),
    ],
)


def _bn_relu(part, xw, selfn, b, g, be):
    h = part[0] + part[1] + selfn[...] * xw[...] + b[...]
    mu = jnp.mean(h, axis=0, keepdims=True)
    var = jnp.mean((h - mu) ** 2, axis=0, keepdims=True)
    h = (h - mu) * lax.rsqrt(var + EPS) * g[...] + be[...]
    return jnp.maximum(h, 0.0)


def _post_body(part, xw, selfn, b, g, be, wn, xwn_o):
    h = _bn_relu(part, xw, selfn, b, g, be)
    xwn_o[...] = _dot(h, wn[...])


_post_call = pl.pallas_call(
    _post_body,
    out_shape=jax.ShapeDtypeStruct((N, HID), _f32),
)


def _final_body(part, xw, selfn, b, g, be, batch2, wf1, bf1, wf2, bf2, out_o):
    h = _bn_relu(part, xw, selfn, b, g, be)
    bt = batch2[...]
    gids = lax.broadcasted_iota(_i32, (1, G), 1)
    onehot = (bt == gids).astype(_f32)
    ssum = lax.dot_general(onehot, h, (((0,), (0,)), ((), ())),
                           precision=lax.Precision.HIGHEST,
                           preferred_element_type=_f32)
    cnt = jnp.sum(onehot, axis=0)[:, None]
    mean = ssum / jnp.maximum(cnt, 1.0)

    rid = lax.broadcasted_iota(_i32, (G, 1), 0)

    def mbody(gi, sm):
        m = jnp.max(jnp.where(bt == gi, h, -jnp.inf), axis=0)
        return jnp.where(rid == gi, m[None, :], sm)

    smax = lax.fori_loop(0, G, mbody, jnp.full((G, HID), -jnp.inf, _f32))
    smax = jnp.where(cnt > 0, smax, 0.0)

    gf = jnp.concatenate([mean, smax], axis=1)
    r = jnp.maximum(_dot(gf, wf1[...]) + bf1[...], 0.0)
    out_o[...] = _dot(r, wf2[...]) + bf2[...]


_final_call = pl.pallas_call(
    _final_body,
    out_shape=jax.ShapeDtypeStruct((G, 2), _f32),
)


def kernel(x, edge_index, edge_attr, batch, W1, b1, W2, b2, W3, b3,
           g1, be1, g2, be2, g3, be3, Wf1, bf1, Wf2, bf2):
    row = edge_index[0]
    col = edge_index[1]
    ew = edge_attr[:, 0]

    row3 = row.reshape(NW, NCH, CH)
    col3 = col.reshape(NW, NCH, CH)

    zH = jnp.zeros((N, HID), _f32)
    zD = jnp.zeros((N, DW), _f32)

    degp = _deg_call(col3, ew, zD)
    dis, selfn, xw = _pre_call(degp, x, W1)
    nrm = _norm_call(row, col, ew, dis.reshape(-1))

    b1r, b2r, b3r = b1[None], b2[None], b3[None]
    g1r, g2r, g3r = g1[None], g2[None], g3[None]
    be1r, be2r, be3r = be1[None], be2[None], be3[None]

    part = _agg_call(row3, col3, nrm, xw, zH)
    xw = _post_call(part, xw, selfn, b1r, g1r, be1r, W2)
    part = _agg_call(row3, col3, nrm, xw, zH)
    xw = _post_call(part, xw, selfn, b2r, g2r, be2r, W3)
    part = _agg_call(row3, col3, nrm, xw, zH)
    out = _final_call(part, xw, selfn, b3r, g3r, be3r,
                      batch[:, None], Wf1, bf1[None], Wf2, bf2[None])
    return out


# speedup vs baseline: 3.3101x; 3.3101x over previous
import functools

import jax
import jax.numpy as jnp
from jax import lax
from jax.experimental import pallas as pl
from jax.experimental.pallas import tpu as pltpu
from jax.experimental.pallas import tpu_sc as plsc

N = 10000
E = 320000
F_IN = 128
HID = 64
G = 64
EPS = 1e-5

NC = 2
NS = 16
NW = NC * NS
EPW = E // NW
CH = 80
NCH = EPW // CH
NB = 5
RPT = N // NS
DW = 16

_mesh = functools.partial(
    plsc.VectorSubcoreMesh, core_axis_name="c", subcore_axis_name="s")

_f32 = jnp.float32
_i32 = jnp.int32


def _wid():
    return lax.axis_index("c") * NS + lax.axis_index("s")


def _deg_body(col_hbm, ew_hbm, z_hbm, out_hbm, col_s, ew_s, upd, deg_sh):
    cid = lax.axis_index("c")
    sid = lax.axis_index("s")
    wid = cid * NS + sid

    rsl = pl.ds(sid * RPT, RPT)
    pltpu.sync_copy(z_hbm.at[rsl], deg_sh.at[rsl])
    pltpu.sync_copy(col_hbm.at[wid], col_s)
    pltpu.sync_copy(ew_hbm.at[pl.ds(wid * EPW, EPW)], ew_s)
    plsc.subcore_barrier()

    def chunk(j, carry):
        def bld(g2, c2):
            n16 = ew_s[pl.ds(j * CH + g2 * 16, 16)]
            for lane in range(16):
                upd[g2 * 16 + lane, :] = jnp.full((DW,), n16[lane], _f32)
            return c2

        lax.fori_loop(0, CH // 16, bld, 0)
        pltpu.sync_copy(upd, deg_sh.at[col_s.at[j]], add=True)
        return carry

    lax.fori_loop(0, NCH, chunk, 0)
    plsc.subcore_barrier()

    @pl.when(sid == 0)
    def _():
        pltpu.sync_copy(deg_sh, out_hbm.at[cid])


_deg_call = pl.kernel(
    _deg_body,
    out_type=jax.ShapeDtypeStruct((NC, N, DW), _f32),
    mesh=_mesh(),
    compiler_params=pltpu.CompilerParams(use_tc_tiling_on_sc=False, needs_layout_passes=False),
    scratch_types=[
        pltpu.VMEM((NCH, CH), _i32),
        pltpu.VMEM((EPW,), _f32),
        pltpu.VMEM((CH, DW), _f32),
        pltpu.VMEM_SHARED((N, DW), _f32),
    ],
)


def _norm_body(row_hbm, col_hbm, ew_hbm, dis_hbm, out_hbm,
               row_s, col_s, ew_s, dis_s, nrm_s):
    wid = _wid()
    base = wid * EPW
    pltpu.sync_copy(row_hbm.at[pl.ds(base, EPW)], row_s)
    pltpu.sync_copy(col_hbm.at[pl.ds(base, EPW)], col_s)
    pltpu.sync_copy(ew_hbm.at[pl.ds(base, EPW)], ew_s)
    pltpu.sync_copy(dis_hbm, dis_s)

    def body(i, carry):
        sl = pl.ds(i * 16, 16)
        r = row_s[sl]
        c = col_s[sl]
        e = ew_s[sl]
        dr = plsc.load_gather(dis_s, [r])
        dc = plsc.load_gather(dis_s, [c])
        nrm_s[sl] = dr * e * dc
        return carry

    lax.fori_loop(0, EPW // 16, body, 0)
    pltpu.sync_copy(nrm_s, out_hbm.at[pl.ds(base, EPW)])


_norm_call = pl.kernel(
    _norm_body,
    out_type=jax.ShapeDtypeStruct((E,), _f32),
    mesh=_mesh(),
    compiler_params=pltpu.CompilerParams(use_tc_tiling_on_sc=False, needs_layout_passes=False),
    scratch_types=[
        pltpu.VMEM((EPW,), _i32),
        pltpu.VMEM((EPW,), _i32),
        pltpu.VMEM((EPW,), _f32),
        pltpu.VMEM((N,), _f32),
        pltpu.VMEM((EPW,), _f32),
    ],
)


def _agg_body(row_hbm, col_hbm, nrm_hbm, xw_hbm, z_hbm, out_hbm,
              row_s, col_s, nrm_s, rows_v, agg_sh, gsem, ssem):
    cid = lax.axis_index("c")
    sid = lax.axis_index("s")
    wid = cid * NS + sid

    rsl = pl.ds(sid * RPT, RPT)
    pltpu.sync_copy(z_hbm.at[rsl], agg_sh.at[rsl])
    pltpu.sync_copy(row_hbm.at[wid], row_s)
    pltpu.sync_copy(col_hbm.at[wid], col_s)
    pltpu.sync_copy(nrm_hbm.at[pl.ds(wid * EPW, EPW)], nrm_s)
    plsc.subcore_barrier()

    def group(gi, carry):
        gd = [pltpu.async_copy(xw_hbm.at[row_s.at[gi * NB + b]],
                               rows_v.at[pl.ds(b * CH, CH)], gsem)
              for b in range(NB)]
        sd = []
        for b in range(NB):
            j = gi * NB + b
            gd[b].wait()

            def scale(g2, c2, b=b, j=j):
                n16 = nrm_s[pl.ds(j * CH + g2 * 16, 16)]
                lidx = jnp.zeros((16,), _i32)
                for lane in range(16):
                    e = b * CH + g2 * 16 + lane
                    sv = n16.at[lidx].get(mode="promise_in_bounds")
                    for k in range(HID // 16):
                        sl = pl.ds(k * 16, 16)
                        rows_v[e, sl] = rows_v[e, sl] * sv
                    lidx = lidx + 1
                return c2

            lax.fori_loop(0, CH // 16, scale, 0)
            sd.append(pltpu.async_copy(rows_v.at[pl.ds(b * CH, CH)],
                                       agg_sh.at[col_s.at[j]], ssem, add=True))
        for d in sd:
            d.wait()
        return carry

    lax.fori_loop(0, NCH // NB, group, 0)
    plsc.subcore_barrier()
    pltpu.sync_copy(agg_sh.at[rsl], out_hbm.at[cid].at[rsl])


_agg_call = pl.kernel(
    _agg_body,
    out_type=jax.ShapeDtypeStruct((NC, N, HID), _f32),
    mesh=_mesh(),
    compiler_params=pltpu.CompilerParams(use_tc_tiling_on_sc=False, needs_layout_passes=False),
    scratch_types=[
        pltpu.VMEM((NCH, CH), _i32),
        pltpu.VMEM((NCH, CH), _i32),
        pltpu.VMEM((EPW,), _f32),
        pltpu.VMEM((NB * CH, HID), _f32),
        pltpu.VMEM_SHARED((N, HID), _f32),
        pltpu.SemaphoreType.DMA,
        pltpu.SemaphoreType.DMA,
    ],
)


def _dot(a, b):
    return lax.dot_general(a, b, (((1,), (0,)), ((), ())),
                           precision=lax.Precision.HIGHEST,
                           preferred_element_type=_f32)


def _pre_body(degp, x, w1, dis_o, self_o, xw_o):
    deg = degp[0][:, 0:1] + degp[1][:, 0:1] + 1.0
    dis = jnp.where(deg > 0, lax.rsqrt(jnp.maximum(deg, 1e-12)), 0.0)
    dis_o[...] = dis
    self_o[...] = dis * dis
    xw_o[...] = _dot(x[...], w1[...])


_pre_call = pl.pallas_call(
    _pre_body,
    out_shape=[
        jax.ShapeDtypeStruct((N, 1), _f32),
        jax.ShapeDtypeStruct((N, 1), _f32),
        jax.ShapeDtypeStruct((N, HID), _f32),
    ],
)


def _bn_relu(part, xw, selfn, b, g, be):
    h = part[0] + part[1] + selfn[...] * xw[...] + b[...]
    mu = jnp.mean(h, axis=0, keepdims=True)
    var = jnp.mean((h - mu) ** 2, axis=0, keepdims=True)
    h = (h - mu) * lax.rsqrt(var + EPS) * g[...] + be[...]
    return jnp.maximum(h, 0.0)


def _post_body(part, xw, selfn, b, g, be, wn, xwn_o):
    h = _bn_relu(part, xw, selfn, b, g, be)
    xwn_o[...] = _dot(h, wn[...])


_post_call = pl.pallas_call(
    _post_body,
    out_shape=jax.ShapeDtypeStruct((N, HID), _f32),
)


def _final_body(part, xw, selfn, b, g, be, batch2, wf1, bf1, wf2, bf2, out_o):
    h = _bn_relu(part, xw, selfn, b, g, be)
    bt = batch2[...]
    gids = lax.broadcasted_iota(_i32, (1, G), 1)
    onehot = (bt == gids).astype(_f32)
    ssum = lax.dot_general(onehot, h, (((0,), (0,)), ((), ())),
                           precision=lax.Precision.HIGHEST,
                           preferred_element_type=_f32)
    cnt = jnp.sum(onehot, axis=0)[:, None]
    mean = ssum / jnp.maximum(cnt, 1.0)

    rid = lax.broadcasted_iota(_i32, (G, 1), 0)

    def mbody(gi, sm):
        m = jnp.max(jnp.where(bt == gi, h, -jnp.inf), axis=0)
        return jnp.where(rid == gi, m[None, :], sm)

    smax = lax.fori_loop(0, G, mbody, jnp.full((G, HID), -jnp.inf, _f32))
    smax = jnp.where(cnt > 0, smax, 0.0)

    gf = jnp.concatenate([mean, smax], axis=1)
    r = jnp.maximum(_dot(gf, wf1[...]) + bf1[...], 0.0)
    out_o[...] = _dot(r, wf2[...]) + bf2[...]


_final_call = pl.pallas_call(
    _final_body,
    out_shape=jax.ShapeDtypeStruct((G, 2), _f32),
)


def kernel(x, edge_index, edge_attr, batch, W1, b1, W2, b2, W3, b3,
           g1, be1, g2, be2, g3, be3, Wf1, bf1, Wf2, bf2):
    row = edge_index[0]
    col = edge_index[1]
    ew = edge_attr[:, 0]

    row3 = row.reshape(NW, NCH, CH)
    col3 = col.reshape(NW, NCH, CH)

    zH = jnp.zeros((N, HID), _f32)
    zD = jnp.zeros((N, DW), _f32)

    degp = _deg_call(col3, ew, zD)
    dis, selfn, xw = _pre_call(degp, x, W1)
    nrm = _norm_call(row, col, ew, dis.reshape(-1))

    b1r, b2r, b3r = b1[None], b2[None], b3[None]
    g1r, g2r, g3r = g1[None], g2[None], g3[None]
    be1r, be2r, be3r = be1[None], be2[None], be3[None]

    part = _agg_call(row3, col3, nrm, xw, zH)
    xw = _post_call(part, xw, selfn, b1r, g1r, be1r, W2)
    part = _agg_call(row3, col3, nrm, xw, zH)
    xw = _post_call(part, xw, selfn, b2r, g2r, be2r, W3)
    part = _agg_call(row3, col3, nrm, xw, zH)
    out = _final_call(part, xw, selfn, b3r, g3r, be3r,
                      batch[:, None], Wf1, bf1[None], Wf2, bf2[None])
    return out


# speedup vs baseline: 4.3678x; 1.3196x over previous
import functools

import jax
import jax.numpy as jnp
from jax import lax
from jax.experimental import pallas as pl
from jax.experimental.pallas import tpu as pltpu
from jax.experimental.pallas import tpu_sc as plsc

N = 10000
E = 320000
F_IN = 128
HID = 64
G = 64
EPS = 1e-5

NC = 2
NS = 16
NW = NC * NS
EPW = E // NW
CH = 80
NCH = EPW // CH
NB = 5
RPT = N // NS
DW = 16

_mesh = functools.partial(
    plsc.VectorSubcoreMesh, core_axis_name="c", subcore_axis_name="s")

_f32 = jnp.float32
_i32 = jnp.int32


def _wid():
    return lax.axis_index("c") * NS + lax.axis_index("s")


def _deg_body(col_hbm, ew_hbm, z_hbm, out_hbm, col_s, ew_s, upd, deg_sh):
    cid = lax.axis_index("c")
    sid = lax.axis_index("s")
    wid = cid * NS + sid

    rsl = pl.ds(sid * RPT, RPT)
    pltpu.sync_copy(z_hbm.at[rsl], deg_sh.at[rsl])
    pltpu.sync_copy(col_hbm.at[wid], col_s)
    pltpu.sync_copy(ew_hbm.at[pl.ds(wid * EPW, EPW)], ew_s)
    plsc.subcore_barrier()

    def chunk(j, carry):
        def bld(g2, c2):
            n16 = ew_s[pl.ds(j * CH + g2 * 16, 16)]
            for lane in range(16):
                upd[g2 * 16 + lane, :] = jnp.full((DW,), n16[lane], _f32)
            return c2

        lax.fori_loop(0, CH // 16, bld, 0)
        pltpu.sync_copy(upd, deg_sh.at[col_s.at[j]], add=True)
        return carry

    lax.fori_loop(0, NCH, chunk, 0)
    plsc.subcore_barrier()

    @pl.when(sid == 0)
    def _():
        pltpu.sync_copy(deg_sh, out_hbm.at[cid])


_deg_call = pl.kernel(
    _deg_body,
    out_type=jax.ShapeDtypeStruct((NC, N, DW), _f32),
    mesh=_mesh(),
    compiler_params=pltpu.CompilerParams(use_tc_tiling_on_sc=False, needs_layout_passes=False),
    scratch_types=[
        pltpu.VMEM((NCH, CH), _i32),
        pltpu.VMEM((EPW,), _f32),
        pltpu.VMEM((CH, DW), _f32),
        pltpu.VMEM_SHARED((N, DW), _f32),
    ],
)


def _norm_body(row_hbm, col_hbm, ew_hbm, dis_hbm, out_hbm,
               row_s, col_s, ew_s, dis_s, nrm_s):
    wid = _wid()
    base = wid * EPW
    pltpu.sync_copy(row_hbm.at[pl.ds(base, EPW)], row_s)
    pltpu.sync_copy(col_hbm.at[pl.ds(base, EPW)], col_s)
    pltpu.sync_copy(ew_hbm.at[pl.ds(base, EPW)], ew_s)
    pltpu.sync_copy(dis_hbm, dis_s)

    def body(i, carry):
        sl = pl.ds(i * 16, 16)
        r = row_s[sl]
        c = col_s[sl]
        e = ew_s[sl]
        dr = plsc.load_gather(dis_s, [r])
        dc = plsc.load_gather(dis_s, [c])
        nrm_s[sl] = dr * e * dc
        return carry

    lax.fori_loop(0, EPW // 16, body, 0)
    pltpu.sync_copy(nrm_s, out_hbm.at[pl.ds(base, EPW)])


_norm_call = pl.kernel(
    _norm_body,
    out_type=jax.ShapeDtypeStruct((E,), _f32),
    mesh=_mesh(),
    compiler_params=pltpu.CompilerParams(use_tc_tiling_on_sc=False, needs_layout_passes=False),
    scratch_types=[
        pltpu.VMEM((EPW,), _i32),
        pltpu.VMEM((EPW,), _i32),
        pltpu.VMEM((EPW,), _f32),
        pltpu.VMEM((N,), _f32),
        pltpu.VMEM((EPW,), _f32),
    ],
)


def _agg_body(row_hbm, col_hbm, nrm_hbm, xw_hbm, z_hbm, out_hbm,
              row_s, col_s, nrm_s, rows_v, agg_sh, gsem, ssem):
    cid = lax.axis_index("c")
    sid = lax.axis_index("s")
    wid = cid * NS + sid

    rsl = pl.ds(sid * RPT, RPT)
    pltpu.sync_copy(z_hbm.at[rsl], agg_sh.at[rsl])
    pltpu.sync_copy(row_hbm.at[wid], row_s)
    pltpu.sync_copy(col_hbm.at[wid], col_s)
    pltpu.sync_copy(nrm_hbm.at[pl.ds(wid * EPW, EPW)], nrm_s)
    plsc.subcore_barrier()

    def group(gi, carry):
        gd = [pltpu.async_copy(xw_hbm.at[row_s.at[gi * NB + b]],
                               rows_v.at[pl.ds(b * CH, CH)], gsem)
              for b in range(NB)]
        sd = []
        for b in range(NB):
            j = gi * NB + b
            gd[b].wait()

            for g2 in range(CH // 16):
                n16 = nrm_s[pl.ds(j * CH + g2 * 16, 16)]
                lidx = jnp.zeros((16,), _i32)
                for lane in range(16):
                    e = b * CH + g2 * 16 + lane
                    sv = n16.at[lidx].get(mode="promise_in_bounds")
                    for k in range(HID // 16):
                        sl = pl.ds(k * 16, 16)
                        rows_v[e, sl] = rows_v[e, sl] * sv
                    lidx = lidx + 1
            sd.append(pltpu.async_copy(rows_v.at[pl.ds(b * CH, CH)],
                                       agg_sh.at[col_s.at[j]], ssem, add=True))
        for d in sd:
            d.wait()
        return carry

    lax.fori_loop(0, NCH // NB, group, 0)
    plsc.subcore_barrier()
    pltpu.sync_copy(agg_sh.at[rsl], out_hbm.at[cid].at[rsl])


_agg_call = pl.kernel(
    _agg_body,
    out_type=jax.ShapeDtypeStruct((NC, N, HID), _f32),
    mesh=_mesh(),
    compiler_params=pltpu.CompilerParams(use_tc_tiling_on_sc=False, needs_layout_passes=False),
    scratch_types=[
        pltpu.VMEM((NCH, CH), _i32),
        pltpu.VMEM((NCH, CH), _i32),
        pltpu.VMEM((EPW,), _f32),
        pltpu.VMEM((NB * CH, HID), _f32),
        pltpu.VMEM_SHARED((N, HID), _f32),
        pltpu.SemaphoreType.DMA,
        pltpu.SemaphoreType.DMA,
    ],
)


def _dot(a, b):
    return lax.dot_general(a, b, (((1,), (0,)), ((), ())),
                           precision=lax.Precision.HIGHEST,
                           preferred_element_type=_f32)


def _pre_body(degp, x, w1, dis_o, self_o, xw_o):
    deg = degp[0][:, 0:1] + degp[1][:, 0:1] + 1.0
    dis = jnp.where(deg > 0, lax.rsqrt(jnp.maximum(deg, 1e-12)), 0.0)
    dis_o[...] = dis
    self_o[...] = dis * dis
    xw_o[...] = _dot(x[...], w1[...])


_pre_call = pl.pallas_call(
    _pre_body,
    out_shape=[
        jax.ShapeDtypeStruct((N, 1), _f32),
        jax.ShapeDtypeStruct((N, 1), _f32),
        jax.ShapeDtypeStruct((N, HID), _f32),
    ],
)


def _bn_relu(part, xw, selfn, b, g, be):
    h = part[0] + part[1] + selfn[...] * xw[...] + b[...]
    mu = jnp.mean(h, axis=0, keepdims=True)
    var = jnp.mean((h - mu) ** 2, axis=0, keepdims=True)
    h = (h - mu) * lax.rsqrt(var + EPS) * g[...] + be[...]
    return jnp.maximum(h, 0.0)


def _post_body(part, xw, selfn, b, g, be, wn, xwn_o):
    h = _bn_relu(part, xw, selfn, b, g, be)
    xwn_o[...] = _dot(h, wn[...])


_post_call = pl.pallas_call(
    _post_body,
    out_shape=jax.ShapeDtypeStruct((N, HID), _f32),
)


def _final_body(part, xw, selfn, b, g, be, batch2, wf1, bf1, wf2, bf2, out_o):
    h = _bn_relu(part, xw, selfn, b, g, be)
    bt = batch2[...]
    gids = lax.broadcasted_iota(_i32, (1, G), 1)
    onehot = (bt == gids).astype(_f32)
    ssum = lax.dot_general(onehot, h, (((0,), (0,)), ((), ())),
                           precision=lax.Precision.HIGHEST,
                           preferred_element_type=_f32)
    cnt = jnp.sum(onehot, axis=0)[:, None]
    mean = ssum / jnp.maximum(cnt, 1.0)

    rid = lax.broadcasted_iota(_i32, (G, 1), 0)

    def mbody(gi, sm):
        m = jnp.max(jnp.where(bt == gi, h, -jnp.inf), axis=0)
        return jnp.where(rid == gi, m[None, :], sm)

    smax = lax.fori_loop(0, G, mbody, jnp.full((G, HID), -jnp.inf, _f32))
    smax = jnp.where(cnt > 0, smax, 0.0)

    gf = jnp.concatenate([mean, smax], axis=1)
    r = jnp.maximum(_dot(gf, wf1[...]) + bf1[...], 0.0)
    out_o[...] = _dot(r, wf2[...]) + bf2[...]


_final_call = pl.pallas_call(
    _final_body,
    out_shape=jax.ShapeDtypeStruct((G, 2), _f32),
)


def kernel(x, edge_index, edge_attr, batch, W1, b1, W2, b2, W3, b3,
           g1, be1, g2, be2, g3, be3, Wf1, bf1, Wf2, bf2):
    row = edge_index[0]
    col = edge_index[1]
    ew = edge_attr[:, 0]

    row3 = row.reshape(NW, NCH, CH)
    col3 = col.reshape(NW, NCH, CH)

    zH = jnp.zeros((N, HID), _f32)
    zD = jnp.zeros((N, DW), _f32)

    degp = _deg_call(col3, ew, zD)
    dis, selfn, xw = _pre_call(degp, x, W1)
    nrm = _norm_call(row, col, ew, dis.reshape(-1))

    b1r, b2r, b3r = b1[None], b2[None], b3[None]
    g1r, g2r, g3r = g1[None], g2[None], g3[None]
    be1r, be2r, be3r = be1[None], be2[None], be3[None]

    part = _agg_call(row3, col3, nrm, xw, zH)
    xw = _post_call(part, xw, selfn, b1r, g1r, be1r, W2)
    part = _agg_call(row3, col3, nrm, xw, zH)
    xw = _post_call(part, xw, selfn, b2r, g2r, be2r, W3)
    part = _agg_call(row3, col3, nrm, xw, zH)
    out = _final_call(part, xw, selfn, b3r, g3r, be3r,
                      batch[:, None], Wf1, bf1[None], Wf2, bf2[None])
    return out


# speedup vs baseline: 4.5512x; 1.0420x over previous
import functools

import jax
import jax.numpy as jnp
from jax import lax
from jax.experimental import pallas as pl
from jax.experimental.pallas import tpu as pltpu
from jax.experimental.pallas import tpu_sc as plsc

N = 10000
E = 320000
F_IN = 128
HID = 64
G = 64
EPS = 1e-5

NC = 2
NS = 16
NW = NC * NS
EPW = E // NW
CH = 80
NCH = EPW // CH
NB = 5
RPT = N // NS
DW = 16

_mesh = functools.partial(
    plsc.VectorSubcoreMesh, core_axis_name="c", subcore_axis_name="s")

_f32 = jnp.float32
_i32 = jnp.int32


def _wid():
    return lax.axis_index("c") * NS + lax.axis_index("s")


def _agg_body(row_hbm, col_hbm, nrm_hbm, xw_hbm, z_hbm, out_hbm,
              row_s, col_s, nrm_s, rows_v, agg_sh, gsem, ssem):
    cid = lax.axis_index("c")
    sid = lax.axis_index("s")
    wid = cid * NS + sid

    rsl = pl.ds(sid * RPT, RPT)
    pltpu.sync_copy(z_hbm.at[rsl], agg_sh.at[rsl])
    pltpu.sync_copy(row_hbm.at[wid], row_s)
    pltpu.sync_copy(col_hbm.at[wid], col_s)
    pltpu.sync_copy(nrm_hbm.at[pl.ds(wid * EPW, EPW)], nrm_s)
    plsc.subcore_barrier()
    _edge_groups(xw_hbm, row_s, col_s, nrm_s, rows_v, agg_sh, gsem, ssem)
    plsc.subcore_barrier()
    pltpu.sync_copy(agg_sh.at[rsl], out_hbm.at[cid].at[rsl])


def _edge_groups(xw_hbm, row_s, col_s, nrm_s, rows_v, agg_sh, gsem, ssem):
    def group(gi, carry):
        gd = [pltpu.async_copy(xw_hbm.at[row_s.at[gi * NB + b]],
                               rows_v.at[pl.ds(b * CH, CH)], gsem)
              for b in range(NB)]
        sd = []
        for b in range(NB):
            j = gi * NB + b
            gd[b].wait()

            for g2 in range(CH // 16):
                n16 = nrm_s[pl.ds(j * CH + g2 * 16, 16)]
                lidx = jnp.zeros((16,), _i32)
                for lane in range(16):
                    e = b * CH + g2 * 16 + lane
                    sv = n16.at[lidx].get(mode="promise_in_bounds")
                    for k in range(HID // 16):
                        sl = pl.ds(k * 16, 16)
                        rows_v[e, sl] = rows_v[e, sl] * sv
                    lidx = lidx + 1
            sd.append(pltpu.async_copy(rows_v.at[pl.ds(b * CH, CH)],
                                       agg_sh.at[col_s.at[j]], ssem, add=True))
        for d in sd:
            d.wait()
        return carry

    lax.fori_loop(0, NCH // NB, group, 0)


_agg_call = pl.kernel(
    _agg_body,
    out_type=jax.ShapeDtypeStruct((NC, N, HID), _f32),
    mesh=_mesh(),
    compiler_params=pltpu.CompilerParams(use_tc_tiling_on_sc=False, needs_layout_passes=False),
    scratch_types=[
        pltpu.VMEM((NCH, CH), _i32),
        pltpu.VMEM((NCH, CH), _i32),
        pltpu.VMEM((EPW,), _f32),
        pltpu.VMEM((NB * CH, HID), _f32),
        pltpu.VMEM_SHARED((N, HID), _f32),
        pltpu.SemaphoreType.DMA,
        pltpu.SemaphoreType.DMA,
    ],
)


NP = 640
NR = NP // NS


def _hi(v):
    return lax.shift_right_logical(v, 4)


def _lo(v):
    return jnp.bitwise_and(v, 15)


def _sc1_body(row3_hbm, col3_hbm, ew_hbm, xw_hbm,
              zh_hbm, zp_hbm, idx_hbm,
              part_hbm, nrm_out_hbm, self_hbm,
              row_s, col_s, nrm_s, rows_v,
              deg_v, dis2_s, dtmp, disl, idx_s,
              agg_sh, dsum_sh, dis_sh, gsem, ssem):
    cid = lax.axis_index("c")
    sid = lax.axis_index("s")
    wid = cid * NS + sid

    rsl = pl.ds(sid * RPT, RPT)
    psl = pl.ds(sid * NR, NR)
    pltpu.sync_copy(zh_hbm.at[rsl], agg_sh.at[rsl])
    pltpu.sync_copy(zp_hbm.at[psl], dsum_sh.at[psl])
    pltpu.sync_copy(zp_hbm, deg_v)
    pltpu.sync_copy(idx_hbm, idx_s)
    plsc.subcore_barrier()

    for p in range(2):
        w2 = sid * 2 + p
        pltpu.sync_copy(col3_hbm.at[w2], col_s)
        pltpu.sync_copy(ew_hbm.at[pl.ds(w2 * EPW, EPW)], nrm_s)

        def dacc(j, carry):
            for g2 in range(CH // 16):
                c16 = col_s[j, pl.ds(g2 * 16, 16)]
                w16 = nrm_s[pl.ds(j * CH + g2 * 16, 16)]
                plsc.addupdate_scatter(deg_v, [_hi(c16), _lo(c16)], w16)
            return carry

        lax.fori_loop(0, NCH, dacc, 0)

    for c in range(NP // 128):
        pltpu.sync_copy(deg_v.at[pl.ds(c * 128, 128)],
                        dsum_sh.at[idx_s.at[c]], add=True)
    plsc.subcore_barrier()

    pltpu.sync_copy(dsum_sh.at[psl], dtmp)
    for r in range(NR):
        d = dtmp[r, :] + 1.0
        h = plsc.bitcast(
            jnp.int32(0x5F3759DF)
            - lax.shift_right_logical(plsc.bitcast(d, _i32), 1), _f32)
        for _ in range(3):
            h = h * (1.5 - 0.5 * d * h * h)
        disl[r, :] = h
        dtmp[r, :] = h * h
    pltpu.sync_copy(disl, dis_sh.at[psl])

    @pl.when(cid == 0)
    def _():
        pltpu.sync_copy(dtmp, self_hbm.at[psl])

    plsc.subcore_barrier()
    pltpu.sync_copy(dis_sh, dis2_s)

    pltpu.sync_copy(row3_hbm.at[wid], row_s)
    pltpu.sync_copy(col3_hbm.at[wid], col_s)
    pltpu.sync_copy(ew_hbm.at[pl.ds(wid * EPW, EPW)], nrm_s)

    def nbody(j, carry):
        for g2 in range(CH // 16):
            sl = pl.ds(j * CH + g2 * 16, 16)
            r = row_s[j, pl.ds(g2 * 16, 16)]
            c = col_s[j, pl.ds(g2 * 16, 16)]
            dr = plsc.load_gather(dis2_s, [_hi(r), _lo(r)])
            dc = plsc.load_gather(dis2_s, [_hi(c), _lo(c)])
            nrm_s[sl] = dr * nrm_s[sl] * dc
        return carry

    lax.fori_loop(0, NCH, nbody, 0)
    pltpu.sync_copy(nrm_s, nrm_out_hbm.at[pl.ds(wid * EPW, EPW)])

    _edge_groups(xw_hbm, row_s, col_s, nrm_s, rows_v, agg_sh, gsem, ssem)
    plsc.subcore_barrier()
    pltpu.sync_copy(agg_sh.at[rsl], part_hbm.at[cid].at[rsl])


_sc1_call = pl.kernel(
    _sc1_body,
    out_type=[
        jax.ShapeDtypeStruct((NC, N, HID), _f32),
        jax.ShapeDtypeStruct((E,), _f32),
        jax.ShapeDtypeStruct((NP, 16), _f32),
    ],
    mesh=_mesh(),
    compiler_params=pltpu.CompilerParams(use_tc_tiling_on_sc=False, needs_layout_passes=False),
    scratch_types=[
        pltpu.VMEM((NCH, CH), _i32),
        pltpu.VMEM((NCH, CH), _i32),
        pltpu.VMEM((EPW,), _f32),
        pltpu.VMEM((NB * CH, HID), _f32),
        pltpu.VMEM((NP, 16), _f32),
        pltpu.VMEM((NP, 16), _f32),
        pltpu.VMEM((NR, 16), _f32),
        pltpu.VMEM((NR, 16), _f32),
        pltpu.VMEM((NP // 128, 128), _i32),
        pltpu.VMEM_SHARED((N, HID), _f32),
        pltpu.VMEM_SHARED((NP, 16), _f32),
        pltpu.VMEM_SHARED((NP, 16), _f32),
        pltpu.SemaphoreType.DMA,
        pltpu.SemaphoreType.DMA,
    ],
)


def _dot(a, b):
    return lax.dot_general(a, b, (((1,), (0,)), ((), ())),
                           precision=lax.Precision.HIGHEST,
                           preferred_element_type=_f32)


def _pre_body(x, w1, xw_o):
    xw_o[...] = _dot(x[...], w1[...])


_pre_call = pl.pallas_call(
    _pre_body,
    out_shape=jax.ShapeDtypeStruct((N, HID), _f32),
)


def _bn_relu(part, xw, selfn, b, g, be):
    h = part[0] + part[1] + selfn[...] * xw[...] + b[...]
    mu = jnp.mean(h, axis=0, keepdims=True)
    var = jnp.mean((h - mu) ** 2, axis=0, keepdims=True)
    h = (h - mu) * lax.rsqrt(var + EPS) * g[...] + be[...]
    return jnp.maximum(h, 0.0)


def _post_body(part, xw, selfn, b, g, be, wn, xwn_o):
    h = _bn_relu(part, xw, selfn, b, g, be)
    xwn_o[...] = _dot(h, wn[...])


_post_call = pl.pallas_call(
    _post_body,
    out_shape=jax.ShapeDtypeStruct((N, HID), _f32),
)


def _final_body(part, xw, selfn, b, g, be, batch2, wf1, bf1, wf2, bf2, out_o):
    h = _bn_relu(part, xw, selfn, b, g, be)
    bt = batch2[...]
    gids = lax.broadcasted_iota(_i32, (1, G), 1)
    onehot = (bt == gids).astype(_f32)
    ssum = lax.dot_general(onehot, h, (((0,), (0,)), ((), ())),
                           precision=lax.Precision.HIGHEST,
                           preferred_element_type=_f32)
    cnt = jnp.sum(onehot, axis=0)[:, None]
    mean = ssum / jnp.maximum(cnt, 1.0)

    rid = lax.broadcasted_iota(_i32, (G, 1), 0)

    def mbody(gi, sm):
        m = jnp.max(jnp.where(bt == gi, h, -jnp.inf), axis=0)
        return jnp.where(rid == gi, m[None, :], sm)

    smax = lax.fori_loop(0, G, mbody, jnp.full((G, HID), -jnp.inf, _f32))
    smax = jnp.where(cnt > 0, smax, 0.0)

    gf = jnp.concatenate([mean, smax], axis=1)
    r = jnp.maximum(_dot(gf, wf1[...]) + bf1[...], 0.0)
    out_o[...] = _dot(r, wf2[...]) + bf2[...]


_final_call = pl.pallas_call(
    _final_body,
    out_shape=jax.ShapeDtypeStruct((G, 2), _f32),
)


def kernel(x, edge_index, edge_attr, batch, W1, b1, W2, b2, W3, b3,
           g1, be1, g2, be2, g3, be3, Wf1, bf1, Wf2, bf2):
    row = edge_index[0]
    col = edge_index[1]
    ew = edge_attr[:, 0]

    row3 = row.reshape(NW, NCH, CH)
    col3 = col.reshape(NW, NCH, CH)

    zH = jnp.zeros((N, HID), _f32)
    zp = jnp.zeros((NP, 16), _f32)
    idxin = jnp.arange(NP, dtype=_i32).reshape(NP // 128, 128)

    xw = _pre_call(x, W1)
    part, nrm, selfo = _sc1_call(row3, col3, ew, xw, zH, zp, idxin)
    selfn = selfo.reshape(-1)[:N][:, None]

    b1r, b2r, b3r = b1[None], b2[None], b3[None]
    g1r, g2r, g3r = g1[None], g2[None], g3[None]
    be1r, be2r, be3r = be1[None], be2[None], be3[None]

    xw = _post_call(part, xw, selfn, b1r, g1r, be1r, W2)
    part = _agg_call(row3, col3, nrm, xw, zH)
    xw = _post_call(part, xw, selfn, b2r, g2r, be2r, W3)
    part = _agg_call(row3, col3, nrm, xw, zH)
    out = _final_call(part, xw, selfn, b3r, g3r, be3r,
                      batch[:, None], Wf1, bf1[None], Wf2, bf2[None])
    return out
